# Initial kernel scaffold; baseline (speedup 1.0000x reference)
import jax
import jax.numpy as jnp
from jax.experimental import pallas as pl


def kernel(node_feats, edge_feats, edge_index, params):
    raise NotImplementedError("write your pallas kernel here")


# speedup vs baseline: 1.9854x; 1.9854x over previous
import functools

import jax
import jax.numpy as jnp
from jax.experimental import pallas as pl
from jax.experimental.pallas import tpu as pltpu

F32 = jnp.float32
_PREC = jax.lax.Precision.HIGHEST


def _dot(a, b):
    return jax.lax.dot_general(a, b, (((1,), (0,)), ((), ())),
                               precision=_PREC, preferred_element_type=F32)


def _ab_body(h_ref, ws_ref, wd_ref, a_ref, b_ref):
    h = h_ref[...]
    a_ref[...] = _dot(h, ws_ref[...])
    b_ref[...] = _dot(h, wd_ref[...])


def _ab(h, ws, wd, rows=2000):
    n, d = h.shape
    hd = ws.shape[1]
    return pl.pallas_call(
        _ab_body,
        grid=(n // rows,),
        in_specs=[pl.BlockSpec((rows, d), lambda i: (i, 0)),
                  pl.BlockSpec((d, hd), lambda i: (0, 0)),
                  pl.BlockSpec((d, hd), lambda i: (0, 0))],
        out_specs=[pl.BlockSpec((rows, hd), lambda i: (i, 0)),
                   pl.BlockSpec((rows, hd), lambda i: (i, 0))],
        out_shape=[jax.ShapeDtypeStruct((n, hd), F32)] * 2,
    )(h, ws, wd)


def _edge_body(sab_ref, e_ref, wme_ref, bm_ref, wa_ref, ba_ref, w32_ref, m_ref):
    ce = _dot(e_ref[...], wme_ref[...])
    m = jnp.maximum(sab_ref[...] + ce + bm_ref[...], 0.0)
    logit = jnp.sum(m * wa_ref[...], axis=1, keepdims=True) + ba_ref[...]
    ex = jnp.exp(logit)
    m_ref[...] = m
    pad = jnp.zeros((m.shape[0], 3), F32)
    w32_ref[...] = jnp.concatenate([m * ex, ex, pad], axis=1)


def _edge(sab, e, wme, bm, wa, ba, rows=8000):
    ecnt, hd = sab.shape
    de = e.shape[1]
    w32, m = pl.pallas_call(
        _edge_body,
        grid=(ecnt // rows,),
        in_specs=[pl.BlockSpec((rows, hd), lambda i: (i, 0)),
                  pl.BlockSpec((rows, de), lambda i: (i, 0)),
                  pl.BlockSpec((de, hd), lambda i: (0, 0)),
                  pl.BlockSpec((1, hd), lambda i: (0, 0)),
                  pl.BlockSpec((1, hd), lambda i: (0, 0)),
                  pl.BlockSpec((1, 1), lambda i: (0, 0))],
        out_specs=[pl.BlockSpec((rows, 32), lambda i: (i, 0)),
                   pl.BlockSpec((rows, hd), lambda i: (i, 0))],
        out_shape=[jax.ShapeDtypeStruct((ecnt, 32), F32),
                   jax.ShapeDtypeStruct((ecnt, hd), F32)],
    )(sab, e, wme, bm.reshape(1, -1), wa.reshape(1, -1), ba.reshape(1, 1))
    return w32, m


def _ctx_body(h_ref, t_ref, wc_ref, bc_ref, wnh_ref, wnc_ref, bn_ref, hn_ref):
    t = t_ref[...]
    u = t[:, 0:28]
    den = t[:, 28:29]
    scale = 1.0 / (den + 1e-16)
    ctx = _dot(u * scale, wc_ref[...]) + (den * scale) * bc_ref[...]
    hn = _dot(h_ref[...], wnh_ref[...]) + _dot(ctx, wnc_ref[...]) + bn_ref[...]
    hn_ref[...] = jnp.maximum(hn, 0.0)


def _ctx_update(h, t, wc, bc, wnh, wnc, bn, rows=2000):
    n, d = h.shape
    hd = wc.shape[1]
    return pl.pallas_call(
        _ctx_body,
        grid=(n // rows,),
        in_specs=[pl.BlockSpec((rows, d), lambda i: (i, 0)),
                  pl.BlockSpec((rows, 32), lambda i: (i, 0)),
                  pl.BlockSpec((hd, hd), lambda i: (0, 0)),
                  pl.BlockSpec((1, hd), lambda i: (0, 0)),
                  pl.BlockSpec((d, hd), lambda i: (0, 0)),
                  pl.BlockSpec((hd, hd), lambda i: (0, 0)),
                  pl.BlockSpec((1, hd), lambda i: (0, 0))],
        out_specs=pl.BlockSpec((rows, hd), lambda i: (i, 0)),
        out_shape=jax.ShapeDtypeStruct((n, hd), F32),
    )(h, t, wc, bc.reshape(1, -1), wnh, wnc, bn.reshape(1, -1))


def _readout_body(h_ref, *refs):
    out_ref = refs[-1]
    prefs = refs[:-1]
    h = h_ref[...]
    g = jnp.sum(h, axis=0, keepdims=True)
    for i in range(3):
        w1h, w1g, b1, w2, b2, wr, br = prefs[7 * i:7 * i + 7]
        z = jnp.maximum(_dot(h, w1h[...]) + _dot(g, w1g[...]) + b1[...], 0.0)
        logit = jnp.sum(z * w2[...], axis=1, keepdims=True) + b2[...]
        a = jnp.exp(logit - jnp.max(logit))
        a = a / jnp.sum(a)
        g = jnp.sum(a * (_dot(h, wr[...]) + br[...]), axis=0, keepdims=True)
    wp, bp = prefs[21], prefs[22]
    out_ref[...] = _dot(g, wp[...]) + bp[...]


def _readout(h, readout_params, wp, bp):
    n, hd = h.shape
    mh = readout_params[0]["W1"].shape[1]
    flat = []
    specs = [pl.BlockSpec((n, hd), lambda i: (0, 0))]
    for p in readout_params:
        flat += [p["W1"][:hd], p["W1"][hd:], p["b1"].reshape(1, -1),
                 p["W2"].reshape(1, -1), p["b2"].reshape(1, 1),
                 p["Wr"], p["br"].reshape(1, -1)]
        specs += [pl.BlockSpec((hd, mh), lambda i: (0, 0)),
                  pl.BlockSpec((hd, mh), lambda i: (0, 0)),
                  pl.BlockSpec((1, mh), lambda i: (0, 0)),
                  pl.BlockSpec((1, mh), lambda i: (0, 0)),
                  pl.BlockSpec((1, 1), lambda i: (0, 0)),
                  pl.BlockSpec((hd, hd), lambda i: (0, 0)),
                  pl.BlockSpec((1, hd), lambda i: (0, 0))]
    flat += [wp, bp.reshape(1, 1)]
    specs += [pl.BlockSpec((hd, 1), lambda i: (0, 0)),
              pl.BlockSpec((1, 1), lambda i: (0, 0))]
    return pl.pallas_call(
        _readout_body,
        grid=(1,),
        in_specs=specs,
        out_specs=pl.BlockSpec((1, 1), lambda i: (0, 0)),
        out_shape=jax.ShapeDtypeStruct((1, 1), F32),
    )(h, *flat)


def kernel(node_feats, edge_feats, edge_index, params):
    n = node_feats.shape[0]
    src = edge_index[0]
    dst = edge_index[1]
    h = node_feats
    e = edge_feats
    for p in params["layers"]:
        dn = h.shape[1]
        wm = p["Wm"]
        ws, wd, wme = wm[:dn], wm[dn:2 * dn], wm[2 * dn:]
        a, b = _ab(h, ws, wd)
        sab = jnp.take(a, src, axis=0) + jnp.take(b, dst, axis=0)
        w32, m = _edge(sab, e, wme, p["bm"], p["Wa"], p["ba"])
        t = jax.ops.segment_sum(w32, dst, num_segments=n)
        wn = p["Wn"]
        h = _ctx_update(h, t, p["Wc"], p["bc"], wn[:dn], wn[dn:], p["bn"])
        e = m
    return _readout(h, params["readout"], params["Wp"], params["bp"])


# speedup vs baseline: 4.3030x; 2.1674x over previous
import functools

import jax
import jax.numpy as jnp
from jax import lax
from jax.experimental import pallas as pl
from jax.experimental.pallas import tpu as pltpu
from jax.experimental.pallas import tpu_sc as plsc

F32 = jnp.float32
_PREC = jax.lax.Precision.HIGHEST

_NC = 2
_NS = 16
_NW = _NC * _NS
_CH = 80


def _dot(a, b):
    return jax.lax.dot_general(a, b, (((1,), (0,)), ((), ())),
                               precision=_PREC, preferred_element_type=F32)


def _pad32(w):
    return jnp.pad(w, ((0, 0), (0, 32 - w.shape[1])))


def _ab_body(h_ref, ws_ref, wd_ref, a_ref, b_ref):
    h = h_ref[...]
    a_ref[...] = _dot(h, ws_ref[...])
    b_ref[...] = _dot(h, wd_ref[...])


def _ab(h, ws32, wd32, rows=2000):
    n, d = h.shape
    return pl.pallas_call(
        _ab_body,
        grid=(n // rows,),
        in_specs=[pl.BlockSpec((rows, d), lambda i: (i, 0)),
                  pl.BlockSpec((d, 32), lambda i: (0, 0)),
                  pl.BlockSpec((d, 32), lambda i: (0, 0))],
        out_specs=[pl.BlockSpec((rows, 32), lambda i: (i, 0)),
                   pl.BlockSpec((rows, 32), lambda i: (i, 0))],
        out_shape=[jax.ShapeDtypeStruct((n, 32), F32)] * 2,
    )(h, ws32, wd32)


def _sc_gather(a32, b32, src3, dst3):
    ecnt = src3.shape[0] * src3.shape[1] * _CH
    chunks = ecnt // (_NW * _CH)
    mesh = plsc.VectorSubcoreMesh(core_axis_name="c", subcore_axis_name="s",
                                  num_cores=_NC, num_subcores=_NS)

    @functools.partial(
        pl.kernel,
        out_type=[jax.ShapeDtypeStruct((ecnt, 32), F32)] * 2,
        mesh=mesh,
        scratch_types=[
            pltpu.VMEM((chunks, _CH), jnp.int32),
            pltpu.VMEM((chunks, _CH), jnp.int32),
            pltpu.VMEM((_CH, 32), F32),
            pltpu.VMEM((_CH, 32), F32),
            pltpu.SemaphoreType.DMA,
            pltpu.SemaphoreType.DMA,
        ],
        compiler_params=pltpu.CompilerParams(use_tc_tiling_on_sc=False),
    )
    def k(a_hbm, b_hbm, src_hbm, dst_hbm, ga_hbm, gb_hbm,
          sidx, didx, buf_a, buf_b, sem_a, sem_b):
        wid = lax.axis_index("s") * _NC + lax.axis_index("c")
        row0 = wid * chunks
        pltpu.sync_copy(src_hbm.at[wid], sidx)
        pltpu.sync_copy(dst_hbm.at[wid], didx)

        def body(j, carry):
            ca = pltpu.async_copy(a_hbm.at[sidx.at[j]], buf_a, sem_a)
            cb = pltpu.async_copy(b_hbm.at[didx.at[j]], buf_b, sem_b)
            ca.wait()
            cb.wait()
            base = (row0 + j) * _CH
            pltpu.sync_copy(buf_a, ga_hbm.at[pl.ds(base, _CH)])
            pltpu.sync_copy(buf_b, gb_hbm.at[pl.ds(base, _CH)])
            return carry

        lax.fori_loop(0, chunks, body, 0)

    return k(a32, b32, src3, dst3)


def _sc_scatter(w32, dst3, npad):
    ecnt = dst3.shape[0] * dst3.shape[1] * _CH
    chunks = ecnt // (_NW * _CH)
    nrows = npad // _NS
    mesh = plsc.VectorSubcoreMesh(core_axis_name="c", subcore_axis_name="s",
                                  num_cores=_NC, num_subcores=_NS)

    @functools.partial(
        pl.kernel,
        out_type=jax.ShapeDtypeStruct((_NC, npad, 32), F32),
        mesh=mesh,
        scratch_types=[
            pltpu.VMEM((chunks, _CH), jnp.int32),
            pltpu.VMEM((_CH, 32), F32),
            pltpu.VMEM((nrows, 32), F32),
            pltpu.VMEM_SHARED((npad, 32), F32),
            pltpu.SemaphoreType.DMA,
        ],
        compiler_params=pltpu.CompilerParams(use_tc_tiling_on_sc=False),
    )
    def k(w_hbm, dst_hbm, out_hbm, didx, buf, stage, acc, sem):
        cid = lax.axis_index("c")
        sid = lax.axis_index("s")
        wid = sid * _NC + cid
        row0 = wid * chunks
        pltpu.sync_copy(dst_hbm.at[wid], didx)

        def zero_body(t, carry):
            stage[t, pl.ds(0, 16)] = jnp.zeros((16,), F32)
            stage[t, pl.ds(16, 16)] = jnp.zeros((16,), F32)
            return carry

        lax.fori_loop(0, nrows, zero_body, 0)
        pltpu.sync_copy(stage, acc.at[pl.ds(sid * nrows, nrows)])
        plsc.subcore_barrier()

        def body(j, carry):
            pltpu.sync_copy(w_hbm.at[pl.ds((row0 + j) * _CH, _CH)], buf)
            pltpu.sync_copy(buf, acc.at[didx.at[j]], add=True)
            return carry

        lax.fori_loop(0, chunks, body, 0)
        plsc.subcore_barrier()
        pltpu.sync_copy(acc.at[pl.ds(sid * nrows, nrows)], stage)
        pltpu.sync_copy(stage, out_hbm.at[cid, pl.ds(sid * nrows, nrows)])

    return k(w32, dst3)


def _edge_body(ga_ref, gb_ref, e_ref, wme_ref, bm_ref, wa_ref, ba_ref,
               w32_ref, m_ref):
    ce = _dot(e_ref[...], wme_ref[...])
    m = jnp.maximum(ga_ref[...] + gb_ref[...] + ce + bm_ref[...], 0.0)
    logit = jnp.sum(m * wa_ref[...], axis=1, keepdims=True) + ba_ref[...]
    ex = jnp.exp(logit)
    m_ref[...] = m
    col = lax.broadcasted_iota(jnp.int32, (1, 32), 1)
    oh = jnp.where(col == 28, 1.0, 0.0).astype(F32)
    w32_ref[...] = (m + oh) * ex


def _edge(ga, gb, e, wme32, bm32, wa32, ba, rows=8000):
    ecnt = ga.shape[0]
    de = e.shape[1]
    return pl.pallas_call(
        _edge_body,
        grid=(ecnt // rows,),
        in_specs=[pl.BlockSpec((rows, 32), lambda i: (i, 0)),
                  pl.BlockSpec((rows, 32), lambda i: (i, 0)),
                  pl.BlockSpec((rows, de), lambda i: (i, 0)),
                  pl.BlockSpec((de, 32), lambda i: (0, 0)),
                  pl.BlockSpec((1, 32), lambda i: (0, 0)),
                  pl.BlockSpec((1, 32), lambda i: (0, 0)),
                  pl.BlockSpec((1, 1), lambda i: (0, 0))],
        out_specs=[pl.BlockSpec((rows, 32), lambda i: (i, 0)),
                   pl.BlockSpec((rows, 32), lambda i: (i, 0))],
        out_shape=[jax.ShapeDtypeStruct((ecnt, 32), F32)] * 2,
    )(ga, gb, e, wme32, bm32, wa32, ba.reshape(1, 1))


def _ctx_body(h_ref, t0_ref, t1_ref, wc_ref, bc_ref, wnh_ref, wnc_ref, bn_ref,
              hn_ref):
    t = t0_ref[...] + t1_ref[...]
    u = t[:, 0:28]
    den = t[:, 28:29]
    scale = 1.0 / (den + 1e-16)
    ctx = _dot(u * scale, wc_ref[...]) + (den * scale) * bc_ref[...]
    hn = _dot(h_ref[...], wnh_ref[...]) + _dot(ctx, wnc_ref[...]) + bn_ref[...]
    hn_ref[...] = jnp.maximum(hn, 0.0)


def _ctx_update(h, t, wc, bc, wnh, wnc, bn, rows=2000):
    n, d = h.shape
    hd = wc.shape[1]
    return pl.pallas_call(
        _ctx_body,
        grid=(n // rows,),
        in_specs=[pl.BlockSpec((rows, d), lambda i: (i, 0)),
                  pl.BlockSpec((rows, 32), lambda i: (i, 0)),
                  pl.BlockSpec((rows, 32), lambda i: (i, 0)),
                  pl.BlockSpec((hd, hd), lambda i: (0, 0)),
                  pl.BlockSpec((1, hd), lambda i: (0, 0)),
                  pl.BlockSpec((d, hd), lambda i: (0, 0)),
                  pl.BlockSpec((hd, hd), lambda i: (0, 0)),
                  pl.BlockSpec((1, hd), lambda i: (0, 0))],
        out_specs=pl.BlockSpec((rows, hd), lambda i: (i, 0)),
        out_shape=jax.ShapeDtypeStruct((n, hd), F32),
    )(h, t[0], t[1], wc, bc.reshape(1, -1), wnh, wnc, bn.reshape(1, -1))


def _readout_body(h_ref, *refs):
    out_ref = refs[-1]
    prefs = refs[:-1]
    h = h_ref[...]
    g = jnp.sum(h, axis=0, keepdims=True)
    for i in range(3):
        w1h, w1g, b1, w2, b2, wr, br = prefs[7 * i:7 * i + 7]
        z = jnp.maximum(_dot(h, w1h[...]) + _dot(g, w1g[...]) + b1[...], 0.0)
        logit = jnp.sum(z * w2[...], axis=1, keepdims=True) + b2[...]
        a = jnp.exp(logit - jnp.max(logit))
        a = a / jnp.sum(a)
        g = jnp.sum(a * (_dot(h, wr[...]) + br[...]), axis=0, keepdims=True)
    wp, bp = prefs[21], prefs[22]
    out_ref[...] = _dot(g, wp[...]) + bp[...]


def _readout(h, readout_params, wp, bp):
    n, hd = h.shape
    mh = readout_params[0]["W1"].shape[1]
    flat = []
    specs = [pl.BlockSpec((n, hd), lambda i: (0, 0))]
    for p in readout_params:
        flat += [p["W1"][:hd], p["W1"][hd:], p["b1"].reshape(1, -1),
                 p["W2"].reshape(1, -1), p["b2"].reshape(1, 1),
                 p["Wr"], p["br"].reshape(1, -1)]
        specs += [pl.BlockSpec((hd, mh), lambda i: (0, 0)),
                  pl.BlockSpec((hd, mh), lambda i: (0, 0)),
                  pl.BlockSpec((1, mh), lambda i: (0, 0)),
                  pl.BlockSpec((1, mh), lambda i: (0, 0)),
                  pl.BlockSpec((1, 1), lambda i: (0, 0)),
                  pl.BlockSpec((hd, hd), lambda i: (0, 0)),
                  pl.BlockSpec((1, hd), lambda i: (0, 0))]
    flat += [wp, bp.reshape(1, 1)]
    specs += [pl.BlockSpec((hd, 1), lambda i: (0, 0)),
              pl.BlockSpec((1, 1), lambda i: (0, 0))]
    return pl.pallas_call(
        _readout_body,
        grid=(1,),
        in_specs=specs,
        out_specs=pl.BlockSpec((1, 1), lambda i: (0, 0)),
        out_shape=jax.ShapeDtypeStruct((1, 1), F32),
    )(h, *flat)


def kernel(node_feats, edge_feats, edge_index, params):
    n = node_feats.shape[0]
    npad = ((n + 16 * 8 - 1) // (16 * 8)) * (16 * 8)
    ecnt = edge_index.shape[1]
    src3 = edge_index[0].reshape(_NW, ecnt // (_NW * _CH), _CH)
    dst3 = edge_index[1].reshape(_NW, ecnt // (_NW * _CH), _CH)
    h = node_feats
    e = edge_feats
    for li, p in enumerate(params["layers"]):
        dn = h.shape[1]
        wm = p["Wm"]
        a32, b32 = _ab(h, _pad32(wm[:dn]), _pad32(wm[dn:2 * dn]))
        ga, gb = _sc_gather(a32, b32, src3, dst3)
        w32, m = _edge(ga, gb, e, _pad32(wm[2 * dn:]),
                       _pad32(p["bm"].reshape(1, -1)),
                       _pad32(p["Wa"].reshape(1, -1)), p["ba"])
        t = _sc_scatter(w32, dst3, npad)
        wn = p["Wn"]
        h = _ctx_update(h, t[:, :n], p["Wc"], p["bc"], wn[:dn], wn[dn:], p["bn"])
        e = m
    return _readout(h, params["readout"], params["Wp"], params["bp"])


# speedup vs baseline: 4.6025x; 1.0696x over previous
import functools

import jax
import jax.numpy as jnp
from jax import lax
from jax.experimental import pallas as pl
from jax.experimental.pallas import tpu as pltpu
from jax.experimental.pallas import tpu_sc as plsc

F32 = jnp.float32
_PREC = jax.lax.Precision.HIGHEST

_NC = 2
_NS = 16
_NW = _NC * _NS
_CH = 80


def _dot(a, b):
    return jax.lax.dot_general(a, b, (((1,), (0,)), ((), ())),
                               precision=_PREC, preferred_element_type=F32)


def _pad32(w):
    return jnp.pad(w, ((0, 0), (0, 32 - w.shape[1])))


def _ab_body(h_ref, ws_ref, wd_ref, a_ref, b_ref):
    h = h_ref[...]
    a_ref[...] = _dot(h, ws_ref[...])
    b_ref[...] = _dot(h, wd_ref[...])


def _ab(h, ws32, wd32, rows=2000):
    n, d = h.shape
    return pl.pallas_call(
        _ab_body,
        grid=(n // rows,),
        in_specs=[pl.BlockSpec((rows, d), lambda i: (i, 0)),
                  pl.BlockSpec((d, 32), lambda i: (0, 0)),
                  pl.BlockSpec((d, 32), lambda i: (0, 0))],
        out_specs=[pl.BlockSpec((rows, 32), lambda i: (i, 0)),
                   pl.BlockSpec((rows, 32), lambda i: (i, 0))],
        out_shape=[jax.ShapeDtypeStruct((n, 32), F32)] * 2,
    )(h, ws32, wd32)


_G = 5
_GROWS = _G * _CH


def _sc_gather(a32, b32, src3, dst3):
    ecnt = src3.shape[0] * src3.shape[1] * _CH
    chunks = ecnt // (_NW * _CH)
    ngroups = chunks // _G
    mesh = plsc.VectorSubcoreMesh(core_axis_name="c", subcore_axis_name="s",
                                  num_cores=_NC, num_subcores=_NS)

    @functools.partial(
        pl.kernel,
        out_type=[jax.ShapeDtypeStruct((ecnt, 32), F32)] * 2,
        mesh=mesh,
        scratch_types=[
            pltpu.VMEM((chunks, _CH), jnp.int32),
            pltpu.VMEM((chunks, _CH), jnp.int32),
            pltpu.VMEM((_GROWS, 32), F32),
            pltpu.VMEM((_GROWS, 32), F32),
            pltpu.VMEM((_GROWS, 32), F32),
            pltpu.VMEM((_GROWS, 32), F32),
            [pltpu.SemaphoreType.DMA] * 8,
        ],
        compiler_params=pltpu.CompilerParams(use_tc_tiling_on_sc=False),
    )
    def k(a_hbm, b_hbm, src_hbm, dst_hbm, ga_hbm, gb_hbm,
          sidx, didx, ba0, ba1, bb0, bb1, sems):
        gsa = [sems[0], sems[1]]
        gsb = [sems[2], sems[3]]
        wsa = [sems[4], sems[5]]
        wsb = [sems[6], sems[7]]
        bufa = [ba0, ba1]
        bufb = [bb0, bb1]
        wid = lax.axis_index("s") * _NC + lax.axis_index("c")
        row0 = wid * chunks
        pltpu.sync_copy(src_hbm.at[wid], sidx)
        pltpu.sync_copy(dst_hbm.at[wid], didx)

        def fire_gathers(g, s):
            for i in range(_G):
                pltpu.async_copy(a_hbm.at[sidx.at[g * _G + i]],
                                 bufa[s].at[pl.ds(i * _CH, _CH)], gsa[s])
                pltpu.async_copy(b_hbm.at[didx.at[g * _G + i]],
                                 bufb[s].at[pl.ds(i * _CH, _CH)], gsb[s])

        def drain_gathers(s):
            pltpu.make_async_copy(a_hbm.at[pl.ds(0, _GROWS)], bufa[s], gsa[s]).wait()
            pltpu.make_async_copy(b_hbm.at[pl.ds(0, _GROWS)], bufb[s], gsb[s]).wait()

        def fire_writes(g, s):
            base = (row0 + g * _G) * _CH
            pltpu.async_copy(bufa[s], ga_hbm.at[pl.ds(base, _GROWS)], wsa[s])
            pltpu.async_copy(bufb[s], gb_hbm.at[pl.ds(base, _GROWS)], wsb[s])

        def drain_writes(s):
            pltpu.make_async_copy(ga_hbm.at[pl.ds(0, _GROWS)], bufa[s], wsa[s]).wait()
            pltpu.make_async_copy(gb_hbm.at[pl.ds(0, _GROWS)], bufb[s], wsb[s]).wait()

        npairs = (ngroups - 1) // 2
        fire_gathers(0, 0)
        fire_gathers(1, 1)

        def body(kk, carry):
            g = 2 * kk
            drain_gathers(0)
            fire_writes(g, 0)
            drain_gathers(1)
            drain_writes(0)
            fire_gathers(g + 2, 0)
            fire_writes(g + 1, 1)
            drain_writes(1)

            @pl.when(kk < npairs - 1)
            def _():
                fire_gathers(g + 3, 1)

            return carry

        lax.fori_loop(0, npairs, body, 0)
        drain_gathers(0)
        fire_writes(ngroups - 1, 0)
        drain_writes(0)

    return k(a32, b32, src3, dst3)


def _sc_scatter(w32, dst3, npad):
    ecnt = dst3.shape[0] * dst3.shape[1] * _CH
    chunks = ecnt // (_NW * _CH)
    nrows = npad // _NS
    mesh = plsc.VectorSubcoreMesh(core_axis_name="c", subcore_axis_name="s",
                                  num_cores=_NC, num_subcores=_NS)

    @functools.partial(
        pl.kernel,
        out_type=jax.ShapeDtypeStruct((_NC, npad, 32), F32),
        mesh=mesh,
        scratch_types=[
            pltpu.VMEM((chunks, _CH), jnp.int32),
            pltpu.VMEM((_CH, 32), F32),
            pltpu.VMEM((nrows, 32), F32),
            pltpu.VMEM_SHARED((npad, 32), F32),
        ],
        compiler_params=pltpu.CompilerParams(use_tc_tiling_on_sc=False),
    )
    def k(w_hbm, dst_hbm, out_hbm, didx, rb0, stage, acc):
        cid = lax.axis_index("c")
        sid = lax.axis_index("s")
        wid = sid * _NC + cid
        row0 = wid * chunks
        pltpu.sync_copy(dst_hbm.at[wid], didx)

        def zero_body(t, carry):
            stage[t, pl.ds(0, 16)] = jnp.zeros((16,), F32)
            stage[t, pl.ds(16, 16)] = jnp.zeros((16,), F32)
            return carry

        lax.fori_loop(0, nrows, zero_body, 0)
        pltpu.sync_copy(stage, acc.at[pl.ds(sid * nrows, nrows)])
        plsc.subcore_barrier()

        def body(j, carry):
            pltpu.sync_copy(w_hbm.at[pl.ds((row0 + j) * _CH, _CH)], rb0)
            pltpu.sync_copy(rb0, acc.at[didx.at[j]], add=True)
            return carry

        lax.fori_loop(0, chunks, body, 0)
        plsc.subcore_barrier()
        pltpu.sync_copy(acc.at[pl.ds(sid * nrows, nrows)], stage)
        pltpu.sync_copy(stage, out_hbm.at[cid, pl.ds(sid * nrows, nrows)])

    return k(w32, dst3)


def _edge_body(recon, ga_ref, gb_ref, e_ref, wme_ref, bm_ref, wa_ref, ba_ref,
               w32_ref):
    e = e_ref[...]
    if recon:
        e = e * (1.0 / e[:, 28:29])
    ce = _dot(e, wme_ref[...])
    m = jnp.maximum(ga_ref[...] + gb_ref[...] + ce + bm_ref[...], 0.0)
    logit = jnp.sum(m * wa_ref[...], axis=1, keepdims=True) + ba_ref[...]
    ex = jnp.exp(logit)
    col = lax.broadcasted_iota(jnp.int32, (1, 32), 1)
    oh = jnp.where(col == 28, 1.0, 0.0).astype(F32)
    w32_ref[...] = (m + oh) * ex


def _edge(ga, gb, e, wme32, bm32, wa32, ba, recon, rows=8000):
    ecnt = ga.shape[0]
    de = e.shape[1]
    return pl.pallas_call(
        functools.partial(_edge_body, recon),
        grid=(ecnt // rows,),
        in_specs=[pl.BlockSpec((rows, 32), lambda i: (i, 0)),
                  pl.BlockSpec((rows, 32), lambda i: (i, 0)),
                  pl.BlockSpec((rows, de), lambda i: (i, 0)),
                  pl.BlockSpec((de, 32), lambda i: (0, 0)),
                  pl.BlockSpec((1, 32), lambda i: (0, 0)),
                  pl.BlockSpec((1, 32), lambda i: (0, 0)),
                  pl.BlockSpec((1, 1), lambda i: (0, 0))],
        out_specs=pl.BlockSpec((rows, 32), lambda i: (i, 0)),
        out_shape=jax.ShapeDtypeStruct((ecnt, 32), F32),
    )(ga, gb, e, wme32, bm32, wa32, ba.reshape(1, 1))


def _ctx_body(h_ref, t0_ref, t1_ref, wc_ref, bc_ref, wnh_ref, wnc_ref, bn_ref,
              hn_ref):
    t = t0_ref[...] + t1_ref[...]
    u = t[:, 0:28]
    den = t[:, 28:29]
    scale = 1.0 / (den + 1e-16)
    ctx = _dot(u * scale, wc_ref[...]) + (den * scale) * bc_ref[...]
    hn = _dot(h_ref[...], wnh_ref[...]) + _dot(ctx, wnc_ref[...]) + bn_ref[...]
    hn_ref[...] = jnp.maximum(hn, 0.0)


def _ctx_update(h, t, wc, bc, wnh, wnc, bn, rows=2000):
    n, d = h.shape
    hd = wc.shape[1]
    return pl.pallas_call(
        _ctx_body,
        grid=(n // rows,),
        in_specs=[pl.BlockSpec((rows, d), lambda i: (i, 0)),
                  pl.BlockSpec((rows, 32), lambda i: (i, 0)),
                  pl.BlockSpec((rows, 32), lambda i: (i, 0)),
                  pl.BlockSpec((hd, hd), lambda i: (0, 0)),
                  pl.BlockSpec((1, hd), lambda i: (0, 0)),
                  pl.BlockSpec((d, hd), lambda i: (0, 0)),
                  pl.BlockSpec((hd, hd), lambda i: (0, 0)),
                  pl.BlockSpec((1, hd), lambda i: (0, 0))],
        out_specs=pl.BlockSpec((rows, hd), lambda i: (i, 0)),
        out_shape=jax.ShapeDtypeStruct((n, hd), F32),
    )(h, t[0], t[1], wc, bc.reshape(1, -1), wnh, wnc, bn.reshape(1, -1))


def _readout_body(h_ref, *refs):
    out_ref = refs[-1]
    prefs = refs[:-1]
    h = h_ref[...]
    g = jnp.sum(h, axis=0, keepdims=True)
    for i in range(3):
        w1h, w1g, b1, w2, b2, wr, br = prefs[7 * i:7 * i + 7]
        z = jnp.maximum(_dot(h, w1h[...]) + _dot(g, w1g[...]) + b1[...], 0.0)
        logit = jnp.sum(z * w2[...], axis=1, keepdims=True) + b2[...]
        a = jnp.exp(logit - jnp.max(logit))
        a = a / jnp.sum(a)
        g = jnp.sum(a * (_dot(h, wr[...]) + br[...]), axis=0, keepdims=True)
    wp, bp = prefs[21], prefs[22]
    out_ref[...] = _dot(g, wp[...]) + bp[...]


def _readout(h, readout_params, wp, bp):
    n, hd = h.shape
    mh = readout_params[0]["W1"].shape[1]
    flat = []
    specs = [pl.BlockSpec((n, hd), lambda i: (0, 0))]
    for p in readout_params:
        flat += [p["W1"][:hd], p["W1"][hd:], p["b1"].reshape(1, -1),
                 p["W2"].reshape(1, -1), p["b2"].reshape(1, 1),
                 p["Wr"], p["br"].reshape(1, -1)]
        specs += [pl.BlockSpec((hd, mh), lambda i: (0, 0)),
                  pl.BlockSpec((hd, mh), lambda i: (0, 0)),
                  pl.BlockSpec((1, mh), lambda i: (0, 0)),
                  pl.BlockSpec((1, mh), lambda i: (0, 0)),
                  pl.BlockSpec((1, 1), lambda i: (0, 0)),
                  pl.BlockSpec((hd, hd), lambda i: (0, 0)),
                  pl.BlockSpec((1, hd), lambda i: (0, 0))]
    flat += [wp, bp.reshape(1, 1)]
    specs += [pl.BlockSpec((hd, 1), lambda i: (0, 0)),
              pl.BlockSpec((1, 1), lambda i: (0, 0))]
    return pl.pallas_call(
        _readout_body,
        grid=(1,),
        in_specs=specs,
        out_specs=pl.BlockSpec((1, 1), lambda i: (0, 0)),
        out_shape=jax.ShapeDtypeStruct((1, 1), F32),
    )(h, *flat)


def kernel(node_feats, edge_feats, edge_index, params):
    n = node_feats.shape[0]
    npad = ((n + 16 * 8 - 1) // (16 * 8)) * (16 * 8)
    ecnt = edge_index.shape[1]
    src3 = edge_index[0].reshape(_NW, ecnt // (_NW * _CH), _CH)
    dst3 = edge_index[1].reshape(_NW, ecnt // (_NW * _CH), _CH)
    h = node_feats
    e = edge_feats
    for li, p in enumerate(params["layers"]):
        dn = h.shape[1]
        wm = p["Wm"]
        wme = wm[2 * dn:]
        wme32 = jnp.pad(wme, ((0, e.shape[1] - wme.shape[0]), (0, 32 - wme.shape[1])))
        a32, b32 = _ab(h, _pad32(wm[:dn]), _pad32(wm[dn:2 * dn]))
        ga, gb = _sc_gather(a32, b32, src3, dst3)
        w32 = _edge(ga, gb, e, wme32,
                    _pad32(p["bm"].reshape(1, -1)),
                    _pad32(p["Wa"].reshape(1, -1)), p["ba"], recon=li > 0)
        t = _sc_scatter(w32, dst3, npad)
        wn = p["Wn"]
        h = _ctx_update(h, t[:, :n], p["Wc"], p["bc"], wn[:dn], wn[dn:], p["bn"])
        e = w32
    return _readout(h, params["readout"], params["Wp"], params["bp"])


# speedup vs baseline: 12.0311x; 2.6140x over previous
import functools

import jax
import jax.numpy as jnp
from jax import lax
from jax.experimental import pallas as pl
from jax.experimental.pallas import tpu as pltpu
from jax.experimental.pallas import tpu_sc as plsc

F32 = jnp.float32
_PREC = jax.lax.Precision.HIGHEST

_NC = 2
_NS = 16
_NW = _NC * _NS
_CH = 80


def _dot(a, b):
    return jax.lax.dot_general(a, b, (((1,), (0,)), ((), ())),
                               precision=_PREC, preferred_element_type=F32)


def _pad32(w):
    return jnp.pad(w, ((0, 0), (0, 32 - w.shape[1])))


def _ab_body(h_ref, ws_ref, wd_ref, a_ref, b_ref):
    h = h_ref[...]
    a_ref[...] = _dot(h, ws_ref[...])
    b_ref[...] = _dot(h, wd_ref[...])


def _ab(h, ws32, wd32, rows=2000):
    n, d = h.shape
    return pl.pallas_call(
        _ab_body,
        grid=(n // rows,),
        in_specs=[pl.BlockSpec((rows, d), lambda i: (i, 0)),
                  pl.BlockSpec((d, 32), lambda i: (0, 0)),
                  pl.BlockSpec((d, 32), lambda i: (0, 0))],
        out_specs=[pl.BlockSpec((rows, 32), lambda i: (i, 0)),
                   pl.BlockSpec((rows, 32), lambda i: (i, 0))],
        out_shape=[jax.ShapeDtypeStruct((n, 32), F32)] * 2,
    )(h, ws32, wd32)


_G = 5
_GROWS = _G * _CH


def _sc_gather(a32, b32, src3, dst3):
    ecnt = src3.shape[0] * src3.shape[1] * _CH
    chunks = ecnt // (_NW * _CH)
    ngroups = chunks // _G
    mesh = plsc.VectorSubcoreMesh(core_axis_name="c", subcore_axis_name="s",
                                  num_cores=_NC, num_subcores=_NS)

    @functools.partial(
        pl.kernel,
        out_type=[jax.ShapeDtypeStruct((ecnt, 32), F32)] * 2,
        mesh=mesh,
        scratch_types=[
            pltpu.VMEM((chunks, _CH), jnp.int32),
            pltpu.VMEM((chunks, _CH), jnp.int32),
            pltpu.VMEM((_GROWS, 32), F32),
            pltpu.VMEM((_GROWS, 32), F32),
            pltpu.VMEM((_GROWS, 32), F32),
            pltpu.VMEM((_GROWS, 32), F32),
            [pltpu.SemaphoreType.DMA] * 8,
        ],
        compiler_params=pltpu.CompilerParams(use_tc_tiling_on_sc=False),
    )
    def k(a_hbm, b_hbm, src_hbm, dst_hbm, ga_hbm, gb_hbm,
          sidx, didx, ba0, ba1, bb0, bb1, sems):
        gsa = [sems[0], sems[1]]
        gsb = [sems[2], sems[3]]
        wsa = [sems[4], sems[5]]
        wsb = [sems[6], sems[7]]
        bufa = [ba0, ba1]
        bufb = [bb0, bb1]
        wid = lax.axis_index("s") * _NC + lax.axis_index("c")
        row0 = wid * chunks
        pltpu.sync_copy(src_hbm.at[wid], sidx)
        pltpu.sync_copy(dst_hbm.at[wid], didx)

        def fire_gathers(g, s):
            for i in range(_G):
                pltpu.async_copy(a_hbm.at[sidx.at[g * _G + i]],
                                 bufa[s].at[pl.ds(i * _CH, _CH)], gsa[s])
                pltpu.async_copy(b_hbm.at[didx.at[g * _G + i]],
                                 bufb[s].at[pl.ds(i * _CH, _CH)], gsb[s])

        def drain_gathers(s):
            pltpu.make_async_copy(a_hbm.at[pl.ds(0, _GROWS)], bufa[s], gsa[s]).wait()
            pltpu.make_async_copy(b_hbm.at[pl.ds(0, _GROWS)], bufb[s], gsb[s]).wait()

        def fire_writes(g, s):
            base = (row0 + g * _G) * _CH
            pltpu.async_copy(bufa[s], ga_hbm.at[pl.ds(base, _GROWS)], wsa[s])
            pltpu.async_copy(bufb[s], gb_hbm.at[pl.ds(base, _GROWS)], wsb[s])

        def drain_writes(s):
            pltpu.make_async_copy(ga_hbm.at[pl.ds(0, _GROWS)], bufa[s], wsa[s]).wait()
            pltpu.make_async_copy(gb_hbm.at[pl.ds(0, _GROWS)], bufb[s], wsb[s]).wait()

        npairs = (ngroups - 1) // 2
        fire_gathers(0, 0)
        fire_gathers(1, 1)

        def body(kk, carry):
            g = 2 * kk
            drain_gathers(0)
            fire_writes(g, 0)
            drain_gathers(1)
            drain_writes(0)
            fire_gathers(g + 2, 0)
            fire_writes(g + 1, 1)
            drain_writes(1)

            @pl.when(kk < npairs - 1)
            def _():
                fire_gathers(g + 3, 1)

            return carry

        lax.fori_loop(0, npairs, body, 0)
        drain_gathers(0)
        fire_writes(ngroups - 1, 0)
        drain_writes(0)

    return k(a32, b32, src3, dst3)


def _sc_scatter(w32, dst3, npad):
    ecnt = dst3.shape[0] * dst3.shape[1] * _CH
    chunks = ecnt // (_NW * _CH)
    nrows = npad // _NS
    mesh = plsc.VectorSubcoreMesh(core_axis_name="c", subcore_axis_name="s",
                                  num_cores=_NC, num_subcores=_NS)

    @functools.partial(
        pl.kernel,
        out_type=jax.ShapeDtypeStruct((_NC, npad, 32), F32),
        mesh=mesh,
        scratch_types=[
            pltpu.VMEM((chunks, _CH), jnp.int32),
            pltpu.VMEM((_GROWS, 32), F32),
            pltpu.VMEM((_GROWS, 32), F32),
            pltpu.VMEM((nrows, 32), F32),
            pltpu.VMEM_SHARED((npad, 32), F32),
            [pltpu.SemaphoreType.DMA] * 2,
        ],
        compiler_params=pltpu.CompilerParams(use_tc_tiling_on_sc=False),
    )
    def k(w_hbm, dst_hbm, out_hbm, didx, rb0, rb1, stage, acc, sems):
        cid = lax.axis_index("c")
        sid = lax.axis_index("s")
        wid = sid * _NC + cid
        row0 = wid * chunks
        ngroups = chunks // _G
        rbuf = [rb0, rb1]
        rsem = [sems[0], sems[1]]
        pltpu.sync_copy(dst_hbm.at[wid], didx)

        def zero_body(t, carry):
            stage[t, pl.ds(0, 16)] = jnp.zeros((16,), F32)
            stage[t, pl.ds(16, 16)] = jnp.zeros((16,), F32)
            return carry

        lax.fori_loop(0, nrows, zero_body, 0)
        pltpu.sync_copy(stage, acc.at[pl.ds(sid * nrows, nrows)])
        plsc.subcore_barrier()

        def fire_read(g, s):
            base = (row0 + g * _G) * _CH
            pltpu.async_copy(w_hbm.at[pl.ds(base, _GROWS)], rbuf[s], rsem[s])

        def drain_read(s):
            pltpu.make_async_copy(w_hbm.at[pl.ds(0, _GROWS)], rbuf[s], rsem[s]).wait()

        def scatter_group(g, s):
            for i in range(_G):
                pltpu.sync_copy(rbuf[s].at[pl.ds(i * _CH, _CH)],
                                acc.at[didx.at[g * _G + i]], add=True)

        npairs = (ngroups - 1) // 2
        fire_read(0, 0)
        fire_read(1, 1)

        def body(kk, carry):
            g = 2 * kk
            drain_read(0)
            scatter_group(g, 0)
            fire_read(g + 2, 0)
            drain_read(1)
            scatter_group(g + 1, 1)

            @pl.when(kk < npairs - 1)
            def _():
                fire_read(g + 3, 1)

            return carry

        lax.fori_loop(0, npairs, body, 0)
        drain_read(0)
        scatter_group(ngroups - 1, 0)
        plsc.subcore_barrier()
        pltpu.sync_copy(acc.at[pl.ds(sid * nrows, nrows)], stage)
        pltpu.sync_copy(stage, out_hbm.at[cid, pl.ds(sid * nrows, nrows)])

    return k(w32, dst3)


def _edge_body(recon, ga_ref, gb_ref, e_ref, wme_ref, bm_ref, wa_ref, ba_ref,
               sel_ref, rep_ref, w32_ref):
    e = e_ref[...]
    if recon:
        d4 = _dot(e, sel_ref[...])
        e = e * _dot(1.0 / d4, rep_ref[...])
    ce = _dot(e, wme_ref[...])
    m = jnp.maximum(ga_ref[...] + gb_ref[...] + ce + bm_ref[...], 0.0)
    logit4 = _dot(m, wa_ref[...]) + ba_ref[...]
    ex4 = jnp.exp(logit4)
    exb = _dot(ex4, rep_ref[...])
    col = lax.broadcasted_iota(jnp.int32, (1, 128), 1)
    oh = jnp.where(col % 32 == 28, 1.0, 0.0).astype(F32)
    w32_ref[...] = (m + oh) * exb


def _edge(ga4, gb4, e4, wme_blk, bm4, wa_blk, ba, sel_blk, rep_blk, recon,
          rows=2000):
    er = ga4.shape[0]
    de = e4.shape[1]
    kw = wme_blk.shape[0]
    return pl.pallas_call(
        functools.partial(_edge_body, recon),
        grid=(er // rows,),
        in_specs=[pl.BlockSpec((rows, 128), lambda i: (i, 0)),
                  pl.BlockSpec((rows, 128), lambda i: (i, 0)),
                  pl.BlockSpec((rows, de), lambda i: (i, 0)),
                  pl.BlockSpec((kw, 128), lambda i: (0, 0)),
                  pl.BlockSpec((1, 128), lambda i: (0, 0)),
                  pl.BlockSpec((128, 4), lambda i: (0, 0)),
                  pl.BlockSpec((1, 1), lambda i: (0, 0)),
                  pl.BlockSpec((128, 4), lambda i: (0, 0)),
                  pl.BlockSpec((4, 128), lambda i: (0, 0))],
        out_specs=pl.BlockSpec((rows, 128), lambda i: (i, 0)),
        out_shape=jax.ShapeDtypeStruct((er, 128), F32),
    )(ga4, gb4, e4, wme_blk, bm4, wa_blk, ba.reshape(1, 1), sel_blk, rep_blk)


def _ctx_body(h_ref, t0_ref, t1_ref, wc_ref, bc_ref, wnh_ref, wnc_ref, bn_ref,
              hn_ref):
    t = t0_ref[...] + t1_ref[...]
    u = t[:, 0:28]
    den = t[:, 28:29]
    scale = 1.0 / (den + 1e-16)
    ctx = _dot(u * scale, wc_ref[...]) + (den * scale) * bc_ref[...]
    hn = _dot(h_ref[...], wnh_ref[...]) + _dot(ctx, wnc_ref[...]) + bn_ref[...]
    hn_ref[...] = jnp.maximum(hn, 0.0)


def _ctx_update(h, t, wc, bc, wnh, wnc, bn, rows=2000):
    n, d = h.shape
    hd = wc.shape[1]
    return pl.pallas_call(
        _ctx_body,
        grid=(n // rows,),
        in_specs=[pl.BlockSpec((rows, d), lambda i: (i, 0)),
                  pl.BlockSpec((rows, 32), lambda i: (i, 0)),
                  pl.BlockSpec((rows, 32), lambda i: (i, 0)),
                  pl.BlockSpec((hd, hd), lambda i: (0, 0)),
                  pl.BlockSpec((1, hd), lambda i: (0, 0)),
                  pl.BlockSpec((d, hd), lambda i: (0, 0)),
                  pl.BlockSpec((hd, hd), lambda i: (0, 0)),
                  pl.BlockSpec((1, hd), lambda i: (0, 0))],
        out_specs=pl.BlockSpec((rows, hd), lambda i: (i, 0)),
        out_shape=jax.ShapeDtypeStruct((n, hd), F32),
    )(h, t[0], t[1], wc, bc.reshape(1, -1), wnh, wnc, bn.reshape(1, -1))


def _readout_body(h_ref, *refs):
    out_ref = refs[-1]
    prefs = refs[:-1]
    h = h_ref[...]
    g = jnp.sum(h, axis=0, keepdims=True)
    for i in range(3):
        w1h, w1g, b1, w2, b2, wr, br = prefs[7 * i:7 * i + 7]
        z = jnp.maximum(_dot(h, w1h[...]) + _dot(g, w1g[...]) + b1[...], 0.0)
        logit = jnp.sum(z * w2[...], axis=1, keepdims=True) + b2[...]
        a = jnp.exp(logit - jnp.max(logit))
        a = a / jnp.sum(a)
        g = jnp.sum(a * (_dot(h, wr[...]) + br[...]), axis=0, keepdims=True)
    wp, bp = prefs[21], prefs[22]
    out_ref[...] = _dot(g, wp[...]) + bp[...]


def _readout(h, readout_params, wp, bp):
    n, hd = h.shape
    mh = readout_params[0]["W1"].shape[1]
    flat = []
    specs = [pl.BlockSpec((n, hd), lambda i: (0, 0))]
    for p in readout_params:
        flat += [p["W1"][:hd], p["W1"][hd:], p["b1"].reshape(1, -1),
                 p["W2"].reshape(1, -1), p["b2"].reshape(1, 1),
                 p["Wr"], p["br"].reshape(1, -1)]
        specs += [pl.BlockSpec((hd, mh), lambda i: (0, 0)),
                  pl.BlockSpec((hd, mh), lambda i: (0, 0)),
                  pl.BlockSpec((1, mh), lambda i: (0, 0)),
                  pl.BlockSpec((1, mh), lambda i: (0, 0)),
                  pl.BlockSpec((1, 1), lambda i: (0, 0)),
                  pl.BlockSpec((hd, hd), lambda i: (0, 0)),
                  pl.BlockSpec((1, hd), lambda i: (0, 0))]
    flat += [wp, bp.reshape(1, 1)]
    specs += [pl.BlockSpec((hd, 1), lambda i: (0, 0)),
              pl.BlockSpec((1, 1), lambda i: (0, 0))]
    return pl.pallas_call(
        _readout_body,
        grid=(1,),
        in_specs=specs,
        out_specs=pl.BlockSpec((1, 1), lambda i: (0, 0)),
        out_shape=jax.ShapeDtypeStruct((1, 1), F32),
    )(h, *flat)


def _blkdiag4(w):
    return jax.scipy.linalg.block_diag(w, w, w, w)


def kernel(node_feats, edge_feats, edge_index, params):
    n = node_feats.shape[0]
    npad = ((n + 16 * 8 - 1) // (16 * 8)) * (16 * 8)
    ecnt = edge_index.shape[1]
    src3 = edge_index[0].reshape(_NW, ecnt // (_NW * _CH), _CH)
    dst3 = edge_index[1].reshape(_NW, ecnt // (_NW * _CH), _CH)
    h = node_feats
    e4 = edge_feats.reshape(ecnt // 4, 4 * edge_feats.shape[1])
    sel = jnp.zeros((32, 1), F32).at[28, 0].set(1.0)
    sel_blk = _blkdiag4(sel)
    rep_blk = _blkdiag4(jnp.ones((1, 32), F32))
    for li, p in enumerate(params["layers"]):
        dn = h.shape[1]
        wm = p["Wm"]
        wme = wm[2 * dn:]
        if li == 0:
            wme32 = _pad32(wme)
        else:
            wme32 = jnp.pad(wme, ((0, 4), (0, 4)))
        wme_blk = _blkdiag4(wme32)
        bm4 = jnp.tile(_pad32(p["bm"].reshape(1, -1)), (1, 4))
        wa_blk = _blkdiag4(_pad32(p["Wa"].reshape(1, -1)).T)
        a32, b32 = _ab(h, _pad32(wm[:dn]), _pad32(wm[dn:2 * dn]))
        ga, gb = _sc_gather(a32, b32, src3, dst3)
        w32p = _edge(ga.reshape(ecnt // 4, 128), gb.reshape(ecnt // 4, 128),
                     e4, wme_blk, bm4, wa_blk, p["ba"], sel_blk, rep_blk,
                     recon=li > 0)
        t = _sc_scatter(w32p.reshape(ecnt, 32), dst3, npad)
        wn = p["Wn"]
        h = _ctx_update(h, t[:, :n], p["Wc"], p["bc"], wn[:dn], wn[dn:], p["bn"])
        e4 = w32p
    return _readout(h, params["readout"], params["Wp"], params["bp"])
